# Initial kernel scaffold; baseline (speedup 1.0000x reference)
import jax
import jax.numpy as jnp
from jax.experimental import pallas as pl


def kernel(x, embs):
    raise NotImplementedError("write your pallas kernel here")


# speedup vs baseline: 1.0939x; 1.0939x over previous
import functools

import jax
import jax.numpy as jnp
from jax import lax
from jax.experimental import pallas as pl
from jax.experimental.pallas import tpu as pltpu
from jax.experimental.pallas import tpu_sc as plsc

NUM_EMBEDDINGS = 1000000
EMBEDDING_DIM = 32
BATCH = 16384
HIST = 50
B = BATCH * HIST

_NC = 2
_NS = 16
_NW = _NC * _NS
_B_PER_W = B // _NW
_CHUNK = 1024
_NCHUNK = _B_PER_W // _CHUNK


@jax.jit
def _gather_sc(idx_flat, embs):
    mesh = plsc.VectorSubcoreMesh(core_axis_name="c", subcore_axis_name="s")

    @functools.partial(
        pl.kernel,
        mesh=mesh,
        out_type=jax.ShapeDtypeStruct((B, EMBEDDING_DIM), jnp.float32),
        scratch_types=[
            pltpu.VMEM((_CHUNK,), jnp.int32),
            pltpu.VMEM((_CHUNK, EMBEDDING_DIM), jnp.float32),
            pltpu.SemaphoreType.DMA,
        ],
        compiler_params=pltpu.CompilerParams(use_tc_tiling_on_sc=False),
    )
    def k(idx_hbm, table_hbm, out_hbm, idx_v, rows_v, sem):
        wid = lax.axis_index("s") * _NC + lax.axis_index("c")
        base = wid * _B_PER_W

        def body(i, carry):
            off = base + i * _CHUNK
            pltpu.sync_copy(idx_hbm.at[pl.ds(off, _CHUNK)], idx_v)
            pltpu.async_copy(table_hbm.at[idx_v], rows_v, sem).wait()
            pltpu.sync_copy(rows_v, out_hbm.at[pl.ds(off, _CHUNK)])
            return carry

        lax.fori_loop(0, _NCHUNK, body, 0)

    return k(idx_flat, embs)


def kernel(x, embs):
    idx_flat = x.reshape(-1).astype(jnp.int32)
    out = _gather_sc(idx_flat, embs)
    return out.reshape(x.shape + (EMBEDDING_DIM,))


# speedup vs baseline: 1.1117x; 1.0163x over previous
import functools

import jax
import jax.numpy as jnp
from jax import lax
from jax.experimental import pallas as pl
from jax.experimental.pallas import tpu as pltpu
from jax.experimental.pallas import tpu_sc as plsc

NUM_EMBEDDINGS = 1000000
EMBEDDING_DIM = 32
BATCH = 16384
HIST = 50
B = BATCH * HIST

_NC = 2
_NS = 16
_NW = _NC * _NS
_B_PER_W = B // _NW
_NBUF = 2
_CHUNK = 1600
_NCHUNK = _B_PER_W // _CHUNK


@jax.jit
def _gather_sc(idx_flat, embs):
    mesh = plsc.VectorSubcoreMesh(core_axis_name="c", subcore_axis_name="s")

    @functools.partial(
        pl.kernel,
        mesh=mesh,
        out_type=jax.ShapeDtypeStruct((B, EMBEDDING_DIM), jnp.float32),
        scratch_types=(
            [pltpu.VMEM((_CHUNK,), jnp.int32)] * _NBUF
            + [pltpu.VMEM((_CHUNK, EMBEDDING_DIM), jnp.float32)] * _NBUF
            + [pltpu.SemaphoreType.DMA] * (3 * _NBUF)
        ),
        compiler_params=pltpu.CompilerParams(use_tc_tiling_on_sc=False),
    )
    def k(idx_hbm, table_hbm, out_hbm, i0, i1, r0, r1,
          si0, si1, sg0, sg1, sw0, sw1):
        idx_v, rows_v = [i0, i1], [r0, r1]
        si, sg, sw = [si0, si1], [sg0, sg1], [sw0, sw1]
        wid = lax.axis_index("s") * _NC + lax.axis_index("c")
        base = wid * _B_PER_W

        def idx_copy(g, b):
            return pltpu.make_async_copy(
                idx_hbm.at[pl.ds(base + g * _CHUNK, _CHUNK)], idx_v[b],
                si[b])

        def gather_copy(b):
            return pltpu.make_async_copy(
                table_hbm.at[idx_v[b]], rows_v[b], sg[b])

        def wb_copy(g, b):
            return pltpu.make_async_copy(
                rows_v[b], out_hbm.at[pl.ds(base + g * _CHUNK, _CHUNK)],
                sw[b])

        for b in range(_NBUF):
            idx_copy(b, b).start()
        for b in range(_NBUF):
            idx_copy(b, b).wait()
            gather_copy(b).start()

        def body(j, carry):
            for b in range(_NBUF):
                g = j * _NBUF + b
                gather_copy(b).wait()
                wb_copy(g, b).start()
                idx_copy(g + _NBUF, b).start()
                idx_copy(g + _NBUF, b).wait()
                wb_copy(g, b).wait()
                gather_copy(b).start()
            return carry

        lax.fori_loop(0, (_NCHUNK - _NBUF) // _NBUF, body, 0)

        for b in range(_NBUF):
            g = _NCHUNK - _NBUF + b
            gather_copy(b).wait()
            wb_copy(g, b).start()
        for b in range(_NBUF):
            g = _NCHUNK - _NBUF + b
            wb_copy(g, b).wait()

    return k(idx_flat, embs)


def kernel(x, embs):
    idx_flat = x.reshape(-1).astype(jnp.int32)
    out = _gather_sc(idx_flat, embs)
    return out.reshape(x.shape + (EMBEDDING_DIM,))


# speedup vs baseline: 1.8021x; 1.6210x over previous
import functools

import jax
import jax.numpy as jnp
from jax import lax
from jax.experimental import pallas as pl
from jax.experimental.pallas import tpu as pltpu
from jax.experimental.pallas import tpu_sc as plsc

NUM_EMBEDDINGS = 1000000
EMBEDDING_DIM = 32
BATCH = 16384
HIST = 50

_NC = 2
_NS = 16
_NW = _NC * _NS
_RPW = BATCH // _NW

_RC = 16
_NCH = _RPW // _RC
_NBUF = 2


@jax.jit
def _impl(x, embs):
    @functools.partial(
        pl.kernel,
        mesh=plsc.VectorSubcoreMesh(core_axis_name="c", subcore_axis_name="s"),
        out_type=jax.ShapeDtypeStruct((BATCH, HIST, EMBEDDING_DIM),
                                      jnp.float32),
        scratch_types=(
            [pltpu.VMEM((_RC, HIST), jnp.int32)] * _NBUF
            + [pltpu.VMEM((_RC, HIST, EMBEDDING_DIM), jnp.float32)] * _NBUF
            + [pltpu.SemaphoreType.DMA] * (3 * _NBUF)
        ),
        compiler_params=pltpu.CompilerParams(use_tc_tiling_on_sc=False),
    )
    def kg(x_hbm, table_hbm, out_hbm, i0, i1, r0, r1,
           si0, si1, sg0, sg1, sw0, sw1):
        idxv, rows = [i0, i1], [r0, r1]
        si, sg, sw = [si0, si1], [sg0, sg1], [sw0, sw1]
        w = lax.axis_index("s") * _NC + lax.axis_index("c")
        base = w * _RPW

        def idx_copy(t, b):
            return pltpu.make_async_copy(
                x_hbm.at[pl.ds(base + t * _RC, _RC), :], idxv[b], si[b])

        def gather_start(b):
            for i in range(_RC):
                pltpu.make_async_copy(
                    table_hbm.at[idxv[b].at[i]], rows[b].at[i], sg[b]).start()

        def gather_wait(b):
            for i in range(_RC):
                pltpu.make_async_copy(
                    table_hbm.at[idxv[b].at[i]], rows[b].at[i], sg[b]).wait()

        def wb_copy(t, b):
            return pltpu.make_async_copy(
                rows[b], out_hbm.at[pl.ds(base + t * _RC, _RC)], sw[b])

        for b in range(_NBUF):
            idx_copy(b, b).start()
        for b in range(_NBUF):
            idx_copy(b, b).wait()
            gather_start(b)

        def body(j, carry):
            for b in range(_NBUF):
                t = j * _NBUF + b
                gather_wait(b)
                wb_copy(t, b).start()
                idx_copy(t + _NBUF, b).start()
                idx_copy(t + _NBUF, b).wait()
                wb_copy(t, b).wait()
                gather_start(b)
            return carry

        lax.fori_loop(0, (_NCH - _NBUF) // _NBUF, body, 0)

        for b in range(_NBUF):
            t = _NCH - _NBUF + b
            gather_wait(b)
            wb_copy(t, b).start()
        for b in range(_NBUF):
            t = _NCH - _NBUF + b
            wb_copy(t, b).wait()

    return kg(x, embs)


def kernel(x, embs):
    return _impl(x, embs)
